# baseline (speedup 1.0000x reference)
import functools

import jax
import jax.numpy as jnp
from jax import lax
from jax.experimental import pallas as pl
from jax.experimental.pallas import tpu as pltpu
from jax.experimental.pallas import tpu_sc as plsc

NC = 2
NS = 16
NW = NC * NS
L = 16
CHUNK = 128
K = 2


def _sc_aggregate(x_pad, src, dst, zeros_blk, compute_deg):
    N_pad, D = x_pad.shape
    n_chunks_total = src.shape[0]
    assert n_chunks_total % NW == 0
    NCH = n_chunks_total // NW
    assert NCH % K == 0
    assert N_pad % NS == 0
    rows_per_tile = N_pad // NS
    assert rows_per_tile % CHUNK == 0

    out_type = [
        jax.ShapeDtypeStruct((NC, N_pad, D), jnp.float32),
        jax.ShapeDtypeStruct((NW, N_pad), jnp.float32),
    ]
    scratch_types = [
        pltpu.VMEM((K, CHUNK), jnp.int32),
        pltpu.VMEM((K, CHUNK), jnp.int32),
        [pltpu.VMEM((CHUNK, D), jnp.float32)] * K,
        pltpu.VMEM((1, N_pad), jnp.float32),
        pltpu.VMEM_SHARED((N_pad, D), jnp.float32),
        [pltpu.SemaphoreType.DMA] * K,
    ]

    mesh = plsc.VectorSubcoreMesh(
        core_axis_name="c", subcore_axis_name="s",
        num_cores=NC, num_subcores=NS)

    @functools.partial(
        pl.kernel, out_type=out_type, mesh=mesh,
        scratch_types=scratch_types,
        compiler_params=pltpu.CompilerParams(needs_layout_passes=False))
    def k(x_hbm, src_hbm, dst_hbm, zeros_hbm,
          aggp_hbm, degp_hbm,
          srcb, dstb, rows, deg_v, agg_s, sems):
        cid = lax.axis_index("c")
        sid = lax.axis_index("s")
        wid = sid * NC + cid

        r0 = sid * rows_per_tile
        n_zchunks = rows_per_tile // CHUNK
        pltpu.sync_copy(zeros_hbm, rows[0])

        def zbody(i, _):
            pltpu.sync_copy(rows[0],
                            agg_s.at[pl.ds(r0 + i * CHUNK, CHUNK), :])
            return 0

        lax.fori_loop(0, n_zchunks, zbody, 0)

        if compute_deg:
            z16 = jnp.zeros((L,), jnp.float32)

            def zdbody(i, _):
                deg_v[0, pl.ds(i * L, L)] = z16
                return 0

            lax.fori_loop(0, N_pad // L, zdbody, 0)
        plsc.subcore_barrier()

        c0 = wid * NCH
        ones16 = jnp.ones((L,), jnp.float32)
        zero16 = jnp.zeros((L,), jnp.int32)

        def body(ib, _):
            blk = c0 + ib * K
            pltpu.sync_copy(src_hbm.at[pl.ds(blk, K), :], srcb)
            pltpu.sync_copy(dst_hbm.at[pl.ds(blk, K), :], dstb)
            for j in range(K):
                pltpu.async_copy(x_hbm.at[srcb.at[j]], rows[j], sems[j])
            for j in range(K):
                pltpu.make_async_copy(x_hbm.at[srcb.at[j]], rows[j],
                                      sems[j]).wait()
                pltpu.sync_copy(rows[j], agg_s.at[dstb.at[j]], add=True)
                if compute_deg:
                    for jj in range(CHUNK // L):
                        idx16 = dstb[j, pl.ds(jj * L, L)]
                        plsc.addupdate_scatter(deg_v, [zero16, idx16],
                                               ones16)
            return 0

        lax.fori_loop(0, NCH // K, body, 0)
        plsc.subcore_barrier()

        def wbody(i, _):
            rr = r0 + i * CHUNK
            pltpu.sync_copy(agg_s.at[pl.ds(rr, CHUNK), :], rows[0])
            pltpu.sync_copy(rows[0], aggp_hbm.at[cid, pl.ds(rr, CHUNK), :])
            return 0

        lax.fori_loop(0, n_zchunks, wbody, 0)
        if compute_deg:
            pltpu.sync_copy(deg_v, degp_hbm.at[pl.ds(wid, 1), :])

    return k(x_pad, src, dst, zeros_blk)


def _tc_dense(aggp, degT, x_pad, WlT, bl, WrT, bn, block_rows=256):
    N_pad, D = x_pad.shape
    assert N_pad % block_rows == 0
    grid = (N_pad // block_rows,)

    if bn is None:
        g2 = jnp.zeros((1, D), jnp.float32)
        b2 = jnp.zeros((1, D), jnp.float32)
    else:
        g2 = bn[0].reshape(1, D)
        b2 = bn[1].reshape(1, D)
    bl2 = bl.reshape(1, D)

    def body(aggp_ref, deg_ref, x_ref, wl_ref, bl_ref, wr_ref, g_ref,
             b_ref, o_ref):
        agg = aggp_ref[0] + aggp_ref[1]
        deg = jnp.sum(deg_ref[...], axis=1, keepdims=True)
        deg = jnp.maximum(deg, 1.0)
        agg = agg / deg
        h = (jnp.dot(agg, wl_ref[...], preferred_element_type=jnp.float32)
             + bl_ref[...]
             + jnp.dot(x_ref[...], wr_ref[...],
                       preferred_element_type=jnp.float32))
        if bn is not None:
            scale = g_ref[...] * lax.rsqrt(jnp.float32(1.0 + 1e-5))
            h = jnp.maximum(h * scale + b_ref[...], 0.0)
        o_ref[...] = h

    R = block_rows
    return pl.pallas_call(
        body,
        grid=grid,
        in_specs=[
            pl.BlockSpec((NC, R, D), lambda i: (0, i, 0)),
            pl.BlockSpec((R, NW), lambda i: (i, 0)),
            pl.BlockSpec((R, D), lambda i: (i, 0)),
            pl.BlockSpec((D, D), lambda i: (0, 0)),
            pl.BlockSpec((1, D), lambda i: (0, 0)),
            pl.BlockSpec((D, D), lambda i: (0, 0)),
            pl.BlockSpec((1, D), lambda i: (0, 0)),
            pl.BlockSpec((1, D), lambda i: (0, 0)),
        ],
        out_specs=pl.BlockSpec((R, D), lambda i: (i, 0)),
        out_shape=jax.ShapeDtypeStruct((N_pad, D), jnp.float32),
    )(aggp, degT, x_pad, WlT, bl2, WrT, g2, b2)


def kernel(x, edge_index, W_l1, b_l1, W_r1, bn1_g, bn1_b, W_l2, b_l2, W_r2):
    N, D = x.shape
    N_pad = ((N + NW * 8 - 1) // (NW * 8)) * (NW * 8)

    E = edge_index.shape[1]
    epw = K * CHUNK * NW
    E_pad = ((E + epw - 1) // epw) * epw
    pad_val = N_pad - 1
    src = jnp.pad(edge_index[0], (0, E_pad - E), constant_values=pad_val)
    dst = jnp.pad(edge_index[1], (0, E_pad - E), constant_values=pad_val)
    src = src.reshape(E_pad // CHUNK, CHUNK)
    dst = dst.reshape(E_pad // CHUNK, CHUNK)
    x_pad = jnp.pad(x, ((0, N_pad - N), (0, 0)))
    zeros_blk = jnp.zeros((CHUNK, D), jnp.float32)

    aggp1, degp = _sc_aggregate(x_pad, src, dst, zeros_blk, True)
    degT = degp.T
    h = _tc_dense(aggp1, degT, x_pad, W_l1.T, b_l1, W_r1.T, (bn1_g, bn1_b))
    aggp2, _ = _sc_aggregate(h, src, dst, zeros_blk, False)
    out = _tc_dense(aggp2, degT, h, W_l2.T, b_l2, W_r2.T, None)
    return out[:N]


# speedup vs baseline: 1.8034x; 1.8034x over previous
import functools

import jax
import jax.numpy as jnp
from jax import lax
from jax.experimental import pallas as pl
from jax.experimental.pallas import tpu as pltpu
from jax.experimental.pallas import tpu_sc as plsc

NC = 2
NS = 16
NW = NC * NS
L = 16
CHUNK = 80
K = 3


def _sc_aggregate(x_pad, src, dst, zeros_blk, compute_deg):
    N_pad, D = x_pad.shape
    E_pad = src.shape[0]
    assert E_pad % (NW * CHUNK) == 0
    NCH = E_pad // (NW * CHUNK)
    assert NCH % K == 0
    assert N_pad % NS == 0
    rows_per_tile = N_pad // NS
    assert rows_per_tile % CHUNK == 0

    out_type = [
        jax.ShapeDtypeStruct((NC, N_pad, D), jnp.float32),
        jax.ShapeDtypeStruct((NW, N_pad), jnp.float32),
    ]
    scratch_types = [
        [pltpu.VMEM((CHUNK,), jnp.int32)] * K,
        [pltpu.VMEM((CHUNK,), jnp.int32)] * K,
        [pltpu.VMEM((CHUNK, D), jnp.float32)] * K,
        pltpu.VMEM((1, N_pad), jnp.float32),
        pltpu.VMEM_SHARED((N_pad, D), jnp.float32),
        pltpu.SemaphoreType.DMA,
        [pltpu.SemaphoreType.DMA] * K,
    ]

    mesh = plsc.VectorSubcoreMesh(
        core_axis_name="c", subcore_axis_name="s",
        num_cores=NC, num_subcores=NS)

    @functools.partial(
        pl.kernel, out_type=out_type, mesh=mesh,
        scratch_types=scratch_types,
        compiler_params=pltpu.CompilerParams(needs_layout_passes=False))
    def k(x_hbm, src_hbm, dst_hbm, zeros_hbm,
          aggp_hbm, degp_hbm,
          srcb, dstb, rows, deg_v, agg_s, semi, sems):
        cid = lax.axis_index("c")
        sid = lax.axis_index("s")
        wid = sid * NC + cid

        r0 = sid * rows_per_tile
        n_zchunks = rows_per_tile // CHUNK
        pltpu.sync_copy(zeros_hbm, rows[0])

        def zbody(i, _):
            pltpu.sync_copy(rows[0],
                            agg_s.at[pl.ds(r0 + i * CHUNK, CHUNK), :])
            return 0

        lax.fori_loop(0, n_zchunks, zbody, 0)

        if compute_deg:
            z16 = jnp.zeros((L,), jnp.float32)

            def zdbody(i, _):
                deg_v[0, pl.ds(i * L, L)] = z16
                return 0

            lax.fori_loop(0, N_pad // L, zdbody, 0)
        plsc.subcore_barrier()

        c0 = wid * NCH * CHUNK
        ones16 = jnp.ones((L,), jnp.float32)
        zero16 = jnp.zeros((L,), jnp.int32)

        def body(ib, _):
            base = c0 + ib * (K * CHUNK)
            for j in range(K):
                pltpu.async_copy(
                    src_hbm.at[pl.ds(base + j * CHUNK, CHUNK)],
                    srcb[j], semi)
                pltpu.async_copy(
                    dst_hbm.at[pl.ds(base + j * CHUNK, CHUNK)],
                    dstb[j], semi)
            for j in range(K):
                pltpu.make_async_copy(
                    src_hbm.at[pl.ds(base, CHUNK)], srcb[j], semi).wait()
                pltpu.make_async_copy(
                    dst_hbm.at[pl.ds(base, CHUNK)], dstb[j], semi).wait()
            for j in range(K):
                pltpu.async_copy(x_hbm.at[srcb[j]], rows[j], sems[j])
            for j in range(K):
                pltpu.make_async_copy(x_hbm.at[srcb[j]], rows[j],
                                      sems[j]).wait()
                pltpu.sync_copy(rows[j], agg_s.at[dstb[j]], add=True)
                if compute_deg:
                    for jj in range(CHUNK // L):
                        idx16 = dstb[j][pl.ds(jj * L, L)]
                        plsc.addupdate_scatter(deg_v, [zero16, idx16],
                                               ones16)
            return 0

        lax.fori_loop(0, NCH // K, body, 0)
        plsc.subcore_barrier()

        def wbody(i, _):
            rr = r0 + i * CHUNK
            pltpu.sync_copy(agg_s.at[pl.ds(rr, CHUNK), :], rows[0])
            pltpu.sync_copy(rows[0], aggp_hbm.at[cid, pl.ds(rr, CHUNK), :])
            return 0

        lax.fori_loop(0, n_zchunks, wbody, 0)
        if compute_deg:
            pltpu.sync_copy(deg_v, degp_hbm.at[pl.ds(wid, 1), :])

    return k(x_pad, src, dst, zeros_blk)


def _tc_dense(aggp, degT, x_pad, WlT, bl, WrT, bn, block_rows=256):
    N_pad, D = x_pad.shape
    assert N_pad % block_rows == 0
    grid = (N_pad // block_rows,)

    if bn is None:
        g2 = jnp.zeros((1, D), jnp.float32)
        b2 = jnp.zeros((1, D), jnp.float32)
    else:
        g2 = bn[0].reshape(1, D)
        b2 = bn[1].reshape(1, D)
    bl2 = bl.reshape(1, D)

    def body(aggp_ref, deg_ref, x_ref, wl_ref, bl_ref, wr_ref, g_ref,
             b_ref, o_ref):
        agg = aggp_ref[0] + aggp_ref[1]
        deg = jnp.sum(deg_ref[...], axis=1, keepdims=True)
        deg = jnp.maximum(deg, 1.0)
        agg = agg / deg
        h = (jnp.dot(agg, wl_ref[...], preferred_element_type=jnp.float32)
             + bl_ref[...]
             + jnp.dot(x_ref[...], wr_ref[...],
                       preferred_element_type=jnp.float32))
        if bn is not None:
            scale = g_ref[...] * lax.rsqrt(jnp.float32(1.0 + 1e-5))
            h = jnp.maximum(h * scale + b_ref[...], 0.0)
        o_ref[...] = h

    R = block_rows
    return pl.pallas_call(
        body,
        grid=grid,
        in_specs=[
            pl.BlockSpec((NC, R, D), lambda i: (0, i, 0)),
            pl.BlockSpec((R, NW), lambda i: (i, 0)),
            pl.BlockSpec((R, D), lambda i: (i, 0)),
            pl.BlockSpec((D, D), lambda i: (0, 0)),
            pl.BlockSpec((1, D), lambda i: (0, 0)),
            pl.BlockSpec((D, D), lambda i: (0, 0)),
            pl.BlockSpec((1, D), lambda i: (0, 0)),
            pl.BlockSpec((1, D), lambda i: (0, 0)),
        ],
        out_specs=pl.BlockSpec((R, D), lambda i: (i, 0)),
        out_shape=jax.ShapeDtypeStruct((N_pad, D), jnp.float32),
    )(aggp, degT, x_pad, WlT, bl2, WrT, g2, b2)


def kernel(x, edge_index, W_l1, b_l1, W_r1, bn1_g, bn1_b, W_l2, b_l2, W_r2):
    N, D = x.shape
    N_pad = ((N + NW * 8 - 1) // (NW * 8)) * (NW * 8)

    E = edge_index.shape[1]
    epw = K * CHUNK * NW
    E_pad = ((E + epw - 1) // epw) * epw
    pad_val = N_pad - 1
    src = jnp.pad(edge_index[0], (0, E_pad - E), constant_values=pad_val)
    dst = jnp.pad(edge_index[1], (0, E_pad - E), constant_values=pad_val)
    x_pad = jnp.pad(x, ((0, N_pad - N), (0, 0)))
    zeros_blk = jnp.zeros((CHUNK, D), jnp.float32)

    aggp1, degp = _sc_aggregate(x_pad, src, dst, zeros_blk, True)
    degT = degp.T
    h = _tc_dense(aggp1, degT, x_pad, W_l1.T, b_l1, W_r1.T, (bn1_g, bn1_b))
    aggp2, _ = _sc_aggregate(h, src, dst, zeros_blk, False)
    out = _tc_dense(aggp2, degT, h, W_l2.T, b_l2, W_r2.T, None)
    return out[:N]


# speedup vs baseline: 2.0757x; 1.1510x over previous
import functools

import jax
import jax.numpy as jnp
from jax import lax
from jax.experimental import pallas as pl
from jax.experimental.pallas import tpu as pltpu
from jax.experimental.pallas import tpu_sc as plsc

NC = 2
NS = 16
NW = NC * NS
L = 16
CHUNK = 80
K = 3


def _sc_aggregate(x_pad, src, dst, zeros_blk, compute_deg):
    N_pad, D = x_pad.shape
    E_pad = src.shape[0] - 2 * K * CHUNK * NW
    assert E_pad % (NW * CHUNK) == 0
    NCH = E_pad // (NW * CHUNK)
    assert NCH % K == 0
    NB = NCH // K
    assert NB % 2 == 0 and NB >= 4
    assert N_pad % NS == 0
    rows_per_tile = N_pad // NS
    assert rows_per_tile % CHUNK == 0

    out_type = [
        jax.ShapeDtypeStruct((NC, N_pad, D), jnp.float32),
        jax.ShapeDtypeStruct((NW, N_pad), jnp.float32),
    ]
    scratch_types = [
        [pltpu.VMEM((CHUNK,), jnp.int32)] * K,
        [pltpu.VMEM((CHUNK,), jnp.int32)] * K,
        [pltpu.VMEM((CHUNK,), jnp.int32)] * K,
        [pltpu.VMEM((CHUNK,), jnp.int32)] * K,
        [pltpu.VMEM((CHUNK, D), jnp.float32)] * K,
        pltpu.VMEM((1, N_pad), jnp.float32),
        pltpu.VMEM_SHARED((N_pad, D), jnp.float32),
        pltpu.SemaphoreType.DMA,
        pltpu.SemaphoreType.DMA,
        [pltpu.SemaphoreType.DMA] * K,
        [pltpu.SemaphoreType.DMA] * K,
    ]

    mesh = plsc.VectorSubcoreMesh(
        core_axis_name="c", subcore_axis_name="s",
        num_cores=NC, num_subcores=NS)

    @functools.partial(
        pl.kernel, out_type=out_type, mesh=mesh,
        scratch_types=scratch_types,
        compiler_params=pltpu.CompilerParams(needs_layout_passes=False))
    def k(x_hbm, src_hbm, dst_hbm, zeros_hbm,
          aggp_hbm, degp_hbm,
          srcA, dstA, srcB, dstB, rows, deg_v, agg_s,
          semiA, semiB, semg, semsc):
        cid = lax.axis_index("c")
        sid = lax.axis_index("s")
        wid = sid * NC + cid

        r0 = sid * rows_per_tile
        n_zchunks = rows_per_tile // CHUNK
        pltpu.sync_copy(zeros_hbm, rows[0])

        def zbody(i, _):
            pltpu.sync_copy(rows[0],
                            agg_s.at[pl.ds(r0 + i * CHUNK, CHUNK), :])
            return 0

        lax.fori_loop(0, n_zchunks, zbody, 0)

        if compute_deg:
            z16 = jnp.zeros((L,), jnp.float32)

            def zdbody(i, _):
                deg_v[0, pl.ds(i * L, L)] = z16
                return 0

            lax.fori_loop(0, N_pad // L, zdbody, 0)
        plsc.subcore_barrier()

        c0 = wid * NCH * CHUNK
        ones16 = jnp.ones((L,), jnp.float32)
        zero16 = jnp.zeros((L,), jnp.int32)

        def fire_idx(ib, sbufs, dbufs, semi):
            for j in range(K):
                off = c0 + (ib * K + j) * CHUNK
                pltpu.async_copy(src_hbm.at[pl.ds(off, CHUNK)],
                                 sbufs[j], semi)
                pltpu.async_copy(dst_hbm.at[pl.ds(off, CHUNK)],
                                 dbufs[j], semi)

        def wait_idx_pair(sbufs, dbufs, semi, j):
            pltpu.make_async_copy(src_hbm.at[pl.ds(c0, CHUNK)],
                                  sbufs[j], semi).wait()
            pltpu.make_async_copy(dst_hbm.at[pl.ds(c0, CHUNK)],
                                  dbufs[j], semi).wait()

        def blockstep(ib, cur, nxt):
            (srcC, dstC, semiC) = cur
            (srcN, dstN, semiN) = nxt
            for j in range(K):
                pltpu.make_async_copy(x_hbm.at[srcC[j]], rows[j],
                                      semg[j]).wait()
                pltpu.async_copy(rows[j], agg_s.at[dstC[j]], semsc[j],
                                 add=True)
                if compute_deg:
                    for jj in range(CHUNK // L):
                        idx16 = dstC[j][pl.ds(jj * L, L)]
                        plsc.addupdate_scatter(deg_v, [zero16, idx16],
                                               ones16)
            for j in range(K):
                pltpu.make_async_copy(rows[j], agg_s.at[dstC[j]],
                                      semsc[j]).wait()
                wait_idx_pair(srcN, dstN, semiN, j)
                pltpu.async_copy(x_hbm.at[srcN[j]], rows[j], semg[j])
            fire_idx(ib + 2, srcC, dstC, semiC)

        A = (srcA, dstA, semiA)
        B = (srcB, dstB, semiB)

        fire_idx(0, srcA, dstA, semiA)
        fire_idx(1, srcB, dstB, semiB)
        for j in range(K):
            wait_idx_pair(srcA, dstA, semiA, j)
            pltpu.async_copy(x_hbm.at[srcA[j]], rows[j], semg[j])

        def body(t, _):
            blockstep(2 * t, A, B)
            blockstep(2 * t + 1, B, A)
            return 0

        lax.fori_loop(0, NB // 2, body, 0)

        for j in range(K):
            pltpu.make_async_copy(x_hbm.at[srcA[j]], rows[j],
                                  semg[j]).wait()
            wait_idx_pair(srcB, dstB, semiB, j)
        plsc.subcore_barrier()

        def wbody(i, _):
            rr = r0 + i * CHUNK
            pltpu.sync_copy(agg_s.at[pl.ds(rr, CHUNK), :], rows[0])
            pltpu.sync_copy(rows[0], aggp_hbm.at[cid, pl.ds(rr, CHUNK), :])
            return 0

        lax.fori_loop(0, n_zchunks, wbody, 0)
        if compute_deg:
            pltpu.sync_copy(deg_v, degp_hbm.at[pl.ds(wid, 1), :])

    return k(x_pad, src, dst, zeros_blk)


def _tc_dense(aggp, degT, x_pad, WlT, bl, WrT, bn, block_rows=256):
    N_pad, D = x_pad.shape
    assert N_pad % block_rows == 0
    grid = (N_pad // block_rows,)

    if bn is None:
        g2 = jnp.zeros((1, D), jnp.float32)
        b2 = jnp.zeros((1, D), jnp.float32)
    else:
        g2 = bn[0].reshape(1, D)
        b2 = bn[1].reshape(1, D)
    bl2 = bl.reshape(1, D)

    def body(aggp_ref, deg_ref, x_ref, wl_ref, bl_ref, wr_ref, g_ref,
             b_ref, o_ref):
        agg = aggp_ref[0] + aggp_ref[1]
        deg = jnp.sum(deg_ref[...], axis=1, keepdims=True)
        deg = jnp.maximum(deg, 1.0)
        agg = agg / deg
        h = (jnp.dot(agg, wl_ref[...], preferred_element_type=jnp.float32)
             + bl_ref[...]
             + jnp.dot(x_ref[...], wr_ref[...],
                       preferred_element_type=jnp.float32))
        if bn is not None:
            scale = g_ref[...] * lax.rsqrt(jnp.float32(1.0 + 1e-5))
            h = jnp.maximum(h * scale + b_ref[...], 0.0)
        o_ref[...] = h

    R = block_rows
    return pl.pallas_call(
        body,
        grid=grid,
        in_specs=[
            pl.BlockSpec((NC, R, D), lambda i: (0, i, 0)),
            pl.BlockSpec((R, NW), lambda i: (i, 0)),
            pl.BlockSpec((R, D), lambda i: (i, 0)),
            pl.BlockSpec((D, D), lambda i: (0, 0)),
            pl.BlockSpec((1, D), lambda i: (0, 0)),
            pl.BlockSpec((D, D), lambda i: (0, 0)),
            pl.BlockSpec((1, D), lambda i: (0, 0)),
            pl.BlockSpec((1, D), lambda i: (0, 0)),
        ],
        out_specs=pl.BlockSpec((R, D), lambda i: (i, 0)),
        out_shape=jax.ShapeDtypeStruct((N_pad, D), jnp.float32),
    )(aggp, degT, x_pad, WlT, bl2, WrT, g2, b2)


def kernel(x, edge_index, W_l1, b_l1, W_r1, bn1_g, bn1_b, W_l2, b_l2, W_r2):
    N, D = x.shape
    N_pad = ((N + NW * 8 - 1) // (NW * 8)) * (NW * 8)

    E = edge_index.shape[1]
    epw = 2 * K * CHUNK * NW
    E_pad = ((E + epw - 1) // epw) * epw
    extra = 2 * K * CHUNK * NW
    pad_val = N_pad - 1
    src = jnp.pad(edge_index[0], (0, E_pad + extra - E),
                  constant_values=pad_val)
    dst = jnp.pad(edge_index[1], (0, E_pad + extra - E),
                  constant_values=pad_val)
    x_pad = jnp.pad(x, ((0, N_pad - N), (0, 0)))
    zeros_blk = jnp.zeros((CHUNK, D), jnp.float32)

    aggp1, degp = _sc_aggregate(x_pad, src, dst, zeros_blk, True)
    degT = degp.T
    h = _tc_dense(aggp1, degT, x_pad, W_l1.T, b_l1, W_r1.T, (bn1_g, bn1_b))
    aggp2, _ = _sc_aggregate(h, src, dst, zeros_blk, False)
    out = _tc_dense(aggp2, degT, h, W_l2.T, b_l2, W_r2.T, None)
    return out[:N]
